# baseline (speedup 1.0000x reference)
import jax
import jax.numpy as jnp
from jax.experimental import pallas as pl
from jax.experimental.pallas import tpu as pltpu

_B, _T, _C = 32, 512, 512
_H, _S = 512, 4
_OUT = _H * _S
_N = _B * _T
_THR = 1.0
_EPS = 1e-5

_RC = 1024
_NRC = _N // _RC
_TC = 32
_NTC = _T // _TC
_HH = _H // 2


def _stats_body(xt_ref, w_ref, cb_ref, st_ref):
    for s in range(_S):
        h = jax.lax.dot_general(
            xt_ref[...], w_ref[s * _H:(s + 1) * _H, :],
            (((1,), (1,)), ((), ())),
            preferred_element_type=jnp.float32) + cb_ref[s]
        st_ref[0, 0, s] = jnp.concatenate(
            [jnp.sum(h, axis=0, keepdims=True),
             jnp.sum(h * h, axis=0, keepdims=True)], axis=0)


def _scan_body(xt_ref, w_ref, cb_ref, st_ref, g_ref, bb_ref, beta_ref,
               spk_ref, mem_ref, cnt_ref, hbuf, mem_s, acc_s):
    tc = pl.program_id(1)
    cb = cb_ref[...]
    for s in range(_S):
        hbuf[s] = jax.lax.dot_general(
            xt_ref[...], w_ref[s], (((1,), (1,)), ((), ())),
            preferred_element_type=jnp.float32) + cb[s]

    beta = beta_ref[0, 0]
    inv_n = jnp.float32(1.0 / _N)
    st = jnp.sum(st_ref[...], axis=0)
    mean = st[:, 0] * inv_n
    var = st[:, 1] * inv_n - mean * mean
    rs = jax.lax.rsqrt(var + _EPS)
    g = g_ref[...]
    bb = bb_ref[...]

    @pl.when(tc == 0)
    def _():
        mem_s[...] = jnp.zeros_like(mem_s)
        acc_s[...] = jnp.zeros_like(acc_s)

    def body(tt, carry):
        mem, acc = carry
        for s in range(_S):
            h = hbuf[s, pl.ds(tt * _B, _B), :]
            hb = ((h - mean[s]) * rs[s]) * g[s] + bb[s]
            reset = (mem > _THR).astype(jnp.float32)
            mem = beta * mem + hb - reset * _THR
            spk = (mem > _THR).astype(jnp.float32)
            spk_ref[tt * _S + s] = spk
            mem_ref[tt * _S + s] = mem
            acc = acc + spk
        return (mem, acc)

    mem1, acc1 = jax.lax.fori_loop(0, _TC, body, (mem_s[...], acc_s[...]))
    mem_s[...] = mem1
    acc_s[...] = acc1

    @pl.when(tc == _NTC - 1)
    def _():
        cnt_ref[0] = acc1


def kernel(x, conv_w, conv_b, gamma, bn_beta, lif_beta):
    xt = x.transpose(1, 0, 2).reshape(_N, _C)

    parts = pl.pallas_call(
        _stats_body,
        grid=(2, _NRC // 2),
        in_specs=[
            pl.BlockSpec((_RC, _C), lambda p, r: (p * (_NRC // 2) + r, 0)),
            pl.BlockSpec((_OUT, _C), lambda p, r: (0, 0)),
            pl.BlockSpec((_S, _H), lambda p, r: (0, 0)),
        ],
        out_specs=pl.BlockSpec((1, 1, _S, 2, _H),
                               lambda p, r: (p, r, 0, 0, 0)),
        out_shape=jax.ShapeDtypeStruct((2, _NRC // 2, _S, 2, _H),
                                       jnp.float32),
        compiler_params=pltpu.CompilerParams(
            dimension_semantics=("parallel", "arbitrary")),
        name="bn_stats",
    )(xt, conv_w, conv_b.reshape(_S, _H))

    parts = parts.reshape(_NRC, _S, 2, _H)
    w4 = conv_w.reshape(_S, _H, _C)
    cb4 = conv_b.reshape(_S, _H)
    g4 = gamma.reshape(_S, _H)
    bb4 = bn_beta.reshape(_S, _H)
    beta2 = jnp.reshape(lif_beta, (1, 1))

    spk_rec, mem_rec, cnt = pl.pallas_call(
        _scan_body,
        grid=(2, _NTC),
        in_specs=[
            pl.BlockSpec((_TC * _B, _C), lambda hh, t: (t, 0)),
            pl.BlockSpec((_S, _HH, _C), lambda hh, t: (0, hh, 0)),
            pl.BlockSpec((_S, _HH), lambda hh, t: (0, hh)),
            pl.BlockSpec((_NRC, _S, 2, _HH), lambda hh, t: (0, 0, 0, hh)),
            pl.BlockSpec((_S, _HH), lambda hh, t: (0, hh)),
            pl.BlockSpec((_S, _HH), lambda hh, t: (0, hh)),
            pl.BlockSpec(memory_space=pltpu.SMEM),
        ],
        out_specs=[
            pl.BlockSpec((_TC * _S, _B, _HH), lambda hh, t: (t, 0, hh)),
            pl.BlockSpec((_TC * _S, _B, _HH), lambda hh, t: (t, 0, hh)),
            pl.BlockSpec((1, _B, _HH), lambda hh, t: (hh, 0, 0)),
        ],
        out_shape=[
            jax.ShapeDtypeStruct((_T * _S, _B, _H), jnp.float32),
            jax.ShapeDtypeStruct((_T * _S, _B, _H), jnp.float32),
            jax.ShapeDtypeStruct((2, _B, _HH), jnp.float32),
        ],
        scratch_shapes=[pltpu.VMEM((_S, _TC * _B, _HH), jnp.float32),
                        pltpu.VMEM((_B, _HH), jnp.float32),
                        pltpu.VMEM((_B, _HH), jnp.float32)],
        compiler_params=pltpu.CompilerParams(
            dimension_semantics=("parallel", "arbitrary"),
            vmem_limit_bytes=50 * 1024 * 1024),
        name="lif_gemm_scan",
    )(xt, w4, cb4, parts, g4, bb4, beta2)

    sum_spks = cnt.sum()
    return spk_rec, mem_rec, sum_spks


# speedup vs baseline: 1.0250x; 1.0250x over previous
import jax
import jax.numpy as jnp
from jax.experimental import pallas as pl
from jax.experimental.pallas import tpu as pltpu

_B, _T, _C = 32, 512, 512
_H, _S = 512, 4
_OUT = _H * _S
_N = _B * _T
_THR = 1.0
_EPS = 1e-5

_RC = 1024
_NRC = _N // _RC
_TC = 32
_NTC = _T // _TC
_HH = _H // 2


def _stats_body(xt_ref, w_ref, cb_ref, st_ref):
    for s in range(_S):
        h = jax.lax.dot_general(
            xt_ref[...], w_ref[s * _H:(s + 1) * _H, :],
            (((1,), (1,)), ((), ())),
            preferred_element_type=jnp.float32) + cb_ref[s]
        st_ref[0, 0, s] = jnp.concatenate(
            [jnp.sum(h, axis=0, keepdims=True),
             jnp.sum(h * h, axis=0, keepdims=True)], axis=0)


def _scan_body(xt_ref, w_ref, cb_ref, st_ref, g_ref, bb_ref, beta_ref,
               spk_ref, mem_ref, cnt_ref, hbuf, mem_s, acc_s):
    tc = pl.program_id(1)
    cb = cb_ref[...]
    for s in range(_S):
        hbuf[s] = jax.lax.dot_general(
            xt_ref[...], w_ref[s], (((1,), (1,)), ((), ())),
            preferred_element_type=jnp.float32) + cb[s]

    beta = beta_ref[0, 0]
    inv_n = jnp.float32(1.0 / _N)
    st = jnp.sum(st_ref[...], axis=0)
    mean = st[:, 0] * inv_n
    var = st[:, 1] * inv_n - mean * mean
    rs = jax.lax.rsqrt(var + _EPS)
    g = g_ref[...]
    bb = bb_ref[...]

    @pl.when(tc == 0)
    def _():
        mem_s[...] = jnp.zeros_like(mem_s)
        acc_s[...] = jnp.zeros_like(acc_s)

    def body(tt, carry):
        mem, spk, acc = carry
        for s in range(_S):
            h = hbuf[s, pl.ds(tt * _B, _B), :]
            hb = ((h - mean[s]) * rs[s]) * g[s] + bb[s]
            mem = beta * mem + hb - spk * _THR
            spk = (mem > _THR).astype(jnp.float32)
            spk_ref[tt * _S + s] = spk
            mem_ref[tt * _S + s] = mem
            acc = acc + spk
        return (mem, spk, acc)

    spk0 = (mem_s[...] > _THR).astype(jnp.float32)
    mem1, _, acc1 = jax.lax.fori_loop(
        0, _TC, body, (mem_s[...], spk0, acc_s[...]))
    mem_s[...] = mem1
    acc_s[...] = acc1

    @pl.when(tc == _NTC - 1)
    def _():
        cnt_ref[0] = acc1


def kernel(x, conv_w, conv_b, gamma, bn_beta, lif_beta):
    xt = x.transpose(1, 0, 2).reshape(_N, _C).astype(jnp.bfloat16)
    wb = conv_w.astype(jnp.bfloat16)

    parts = pl.pallas_call(
        _stats_body,
        grid=(2, _NRC // 2),
        in_specs=[
            pl.BlockSpec((_RC, _C), lambda p, r: (p * (_NRC // 2) + r, 0)),
            pl.BlockSpec((_OUT, _C), lambda p, r: (0, 0)),
            pl.BlockSpec((_S, _H), lambda p, r: (0, 0)),
        ],
        out_specs=pl.BlockSpec((1, 1, _S, 2, _H),
                               lambda p, r: (p, r, 0, 0, 0)),
        out_shape=jax.ShapeDtypeStruct((2, _NRC // 2, _S, 2, _H),
                                       jnp.float32),
        compiler_params=pltpu.CompilerParams(
            dimension_semantics=("parallel", "arbitrary")),
        name="bn_stats",
    )(xt, wb, conv_b.reshape(_S, _H))

    parts = parts.reshape(_NRC, _S, 2, _H)
    w4 = wb.reshape(_S, _H, _C)
    cb4 = conv_b.reshape(_S, _H)
    g4 = gamma.reshape(_S, _H)
    bb4 = bn_beta.reshape(_S, _H)
    beta2 = jnp.reshape(lif_beta, (1, 1))

    spk_rec, mem_rec, cnt = pl.pallas_call(
        _scan_body,
        grid=(2, _NTC),
        in_specs=[
            pl.BlockSpec((_TC * _B, _C), lambda hh, t: (t, 0)),
            pl.BlockSpec((_S, _HH, _C), lambda hh, t: (0, hh, 0)),
            pl.BlockSpec((_S, _HH), lambda hh, t: (0, hh)),
            pl.BlockSpec((_NRC, _S, 2, _HH), lambda hh, t: (0, 0, 0, hh)),
            pl.BlockSpec((_S, _HH), lambda hh, t: (0, hh)),
            pl.BlockSpec((_S, _HH), lambda hh, t: (0, hh)),
            pl.BlockSpec(memory_space=pltpu.SMEM),
        ],
        out_specs=[
            pl.BlockSpec((_TC * _S, _B, _HH), lambda hh, t: (t, 0, hh)),
            pl.BlockSpec((_TC * _S, _B, _HH), lambda hh, t: (t, 0, hh)),
            pl.BlockSpec((1, _B, _HH), lambda hh, t: (hh, 0, 0)),
        ],
        out_shape=[
            jax.ShapeDtypeStruct((_T * _S, _B, _H), jnp.float32),
            jax.ShapeDtypeStruct((_T * _S, _B, _H), jnp.float32),
            jax.ShapeDtypeStruct((2, _B, _HH), jnp.float32),
        ],
        scratch_shapes=[pltpu.VMEM((_S, _TC * _B, _HH), jnp.float32),
                        pltpu.VMEM((_B, _HH), jnp.float32),
                        pltpu.VMEM((_B, _HH), jnp.float32)],
        compiler_params=pltpu.CompilerParams(
            dimension_semantics=("parallel", "arbitrary"),
            vmem_limit_bytes=50 * 1024 * 1024),
        name="lif_gemm_scan",
    )(xt, w4, cb4, parts, g4, bb4, beta2)

    sum_spks = cnt.sum()
    return spk_rec, mem_rec, sum_spks


# speedup vs baseline: 1.0259x; 1.0009x over previous
import jax
import jax.numpy as jnp
from jax.experimental import pallas as pl
from jax.experimental.pallas import tpu as pltpu

_B, _T, _C = 32, 512, 512
_H, _S = 512, 4
_OUT = _H * _S
_N = _B * _T
_THR = 1.0
_EPS = 1e-5

_RC = 1024
_NRC = _N // _RC
_TC = 32
_NTC = _T // _TC
_HH = _H // 2


def _stats_body(xt_ref, w_ref, cb_ref, st_ref):
    for s in range(_S):
        h = jax.lax.dot_general(
            xt_ref[...], w_ref[s * _H:(s + 1) * _H, :],
            (((1,), (1,)), ((), ())),
            preferred_element_type=jnp.float32) + cb_ref[s]
        st_ref[0, 0, s] = jnp.concatenate(
            [jnp.sum(h, axis=0, keepdims=True),
             jnp.sum(h * h, axis=0, keepdims=True)], axis=0)


def _scan_body(xt_ref, w_ref, cb_ref, st_ref, g_ref, bb_ref, beta_ref,
               spk_ref, mem_ref, cnt_ref, hbuf, mem_s, acc_s):
    tc = pl.program_id(1)
    cb = cb_ref[...]
    for s in range(_S):
        hbuf[s] = jax.lax.dot_general(
            xt_ref[...], w_ref[s], (((1,), (1,)), ((), ())),
            preferred_element_type=jnp.float32) + cb[s]

    beta = beta_ref[0, 0]
    inv_n = jnp.float32(1.0 / _N)
    st = jnp.sum(st_ref[...], axis=0)
    mean = st[:, 0] * inv_n
    var = st[:, 1] * inv_n - mean * mean
    rs = jax.lax.rsqrt(var + _EPS)
    g = g_ref[...]
    bb = bb_ref[...]

    @pl.when(tc == 0)
    def _():
        mem_s[...] = jnp.zeros_like(mem_s)
        acc_s[...] = jnp.zeros_like(acc_s)

    def body(tt, carry):
        mem, spk, acc = carry
        for s in range(_S):
            h = hbuf[s, pl.ds(tt * _B, _B), :]
            hb = ((h - mean[s]) * rs[s]) * g[s] + bb[s]
            mem = beta * mem + hb - spk * _THR
            spk = (mem > _THR).astype(jnp.float32)
            spk_ref[tt * _S + s] = spk
            mem_ref[tt * _S + s] = mem
            acc = acc + spk
        return (mem, spk, acc)

    spk0 = (mem_s[...] > _THR).astype(jnp.float32)
    mem1, _, acc1 = jax.lax.fori_loop(
        0, _TC, body, (mem_s[...], spk0, acc_s[...]))
    mem_s[...] = mem1
    acc_s[...] = acc1

    @pl.when(tc == _NTC - 1)
    def _():
        cnt_ref[0] = acc1


def kernel(x, conv_w, conv_b, gamma, bn_beta, lif_beta):
    xt = x.astype(jnp.bfloat16).transpose(1, 0, 2).reshape(_N, _C)
    wb = conv_w.astype(jnp.bfloat16)

    parts = pl.pallas_call(
        _stats_body,
        grid=(2, _NRC // 2),
        in_specs=[
            pl.BlockSpec((_RC, _C), lambda p, r: (p * (_NRC // 2) + r, 0)),
            pl.BlockSpec((_OUT, _C), lambda p, r: (0, 0)),
            pl.BlockSpec((_S, _H), lambda p, r: (0, 0)),
        ],
        out_specs=pl.BlockSpec((1, 1, _S, 2, _H),
                               lambda p, r: (p, r, 0, 0, 0)),
        out_shape=jax.ShapeDtypeStruct((2, _NRC // 2, _S, 2, _H),
                                       jnp.float32),
        compiler_params=pltpu.CompilerParams(
            dimension_semantics=("parallel", "arbitrary")),
        name="bn_stats",
    )(xt, wb, conv_b.reshape(_S, _H))

    parts = parts.reshape(_NRC, _S, 2, _H)
    w4 = wb.reshape(_S, _H, _C)
    cb4 = conv_b.reshape(_S, _H)
    g4 = gamma.reshape(_S, _H)
    bb4 = bn_beta.reshape(_S, _H)
    beta2 = jnp.reshape(lif_beta, (1, 1))

    spk_rec, mem_rec, cnt = pl.pallas_call(
        _scan_body,
        grid=(2, _NTC),
        in_specs=[
            pl.BlockSpec((_TC * _B, _C), lambda hh, t: (t, 0)),
            pl.BlockSpec((_S, _HH, _C), lambda hh, t: (0, hh, 0)),
            pl.BlockSpec((_S, _HH), lambda hh, t: (0, hh)),
            pl.BlockSpec((_NRC, _S, 2, _HH), lambda hh, t: (0, 0, 0, hh)),
            pl.BlockSpec((_S, _HH), lambda hh, t: (0, hh)),
            pl.BlockSpec((_S, _HH), lambda hh, t: (0, hh)),
            pl.BlockSpec(memory_space=pltpu.SMEM),
        ],
        out_specs=[
            pl.BlockSpec((_TC * _S, _B, _HH), lambda hh, t: (t, 0, hh)),
            pl.BlockSpec((_TC * _S, _B, _HH), lambda hh, t: (t, 0, hh)),
            pl.BlockSpec((1, _B, _HH), lambda hh, t: (hh, 0, 0)),
        ],
        out_shape=[
            jax.ShapeDtypeStruct((_T * _S, _B, _H), jnp.float32),
            jax.ShapeDtypeStruct((_T * _S, _B, _H), jnp.float32),
            jax.ShapeDtypeStruct((2, _B, _HH), jnp.float32),
        ],
        scratch_shapes=[pltpu.VMEM((_S, _TC * _B, _HH), jnp.float32),
                        pltpu.VMEM((_B, _HH), jnp.float32),
                        pltpu.VMEM((_B, _HH), jnp.float32)],
        compiler_params=pltpu.CompilerParams(
            dimension_semantics=("parallel", "arbitrary"),
            vmem_limit_bytes=50 * 1024 * 1024),
        name="lif_gemm_scan",
    )(xt, w4, cb4, parts, g4, bb4, beta2)

    sum_spks = cnt.sum()
    return spk_rec, mem_rec, sum_spks


# speedup vs baseline: 1.1409x; 1.1121x over previous
import jax
import jax.numpy as jnp
from jax.experimental import pallas as pl
from jax.experimental.pallas import tpu as pltpu

_B, _T, _C = 32, 512, 512
_H, _S = 512, 4
_OUT = _H * _S
_N = _B * _T
_THR = 1.0
_EPS = 1e-5

_RC = 1024
_NRC = _N // _RC
_TC = 32
_NTC = _T // _TC
_HH = _H // 2


def _stats_body(x_ref, w_ref, cb_ref, st_ref):
    for s in range(_S):
        h = jax.lax.dot_general(
            x_ref[...], w_ref[s * _H:(s + 1) * _H, :],
            (((1,), (1,)), ((), ())),
            preferred_element_type=jnp.float32) + cb_ref[s]
        st_ref[0, 0, s] = jnp.concatenate(
            [jnp.sum(h, axis=0, keepdims=True),
             jnp.sum(h * h, axis=0, keepdims=True)], axis=0)


def _scan_body(x_ref, p_ref, w_ref, cb_ref, st_ref, g_ref, bb_ref, beta_ref,
               spk_ref, mem_ref, cnt_ref, hbuf, mem_s, acc_s):
    tc = pl.program_id(1)
    cb = cb_ref[...]
    xv = x_ref[...].reshape(_B * _TC, _C).astype(jnp.bfloat16)
    xp = jax.lax.dot_general(
        p_ref[...], xv, (((1,), (0,)), ((), ())),
        preferred_element_type=jnp.float32).astype(jnp.bfloat16)
    for s in range(_S):
        hb = jax.lax.dot_general(
            xp, w_ref[s], (((1,), (1,)), ((), ())),
            preferred_element_type=jnp.float32) + cb[s]
        hbuf[s] = hb.reshape(_TC, _B, _HH)

    beta = beta_ref[0, 0]
    inv_n = jnp.float32(1.0 / _N)
    st = jnp.sum(st_ref[...], axis=0)
    mean = st[:, 0] * inv_n
    var = st[:, 1] * inv_n - mean * mean
    rs = jax.lax.rsqrt(var + _EPS)
    g = g_ref[...]
    bb = bb_ref[...]

    @pl.when(tc == 0)
    def _():
        mem_s[...] = jnp.zeros_like(mem_s)
        acc_s[...] = jnp.zeros_like(acc_s)

    def body(tt, carry):
        mem, spk, acc = carry
        for s in range(_S):
            h = hbuf[s, tt]
            hb = ((h - mean[s]) * rs[s]) * g[s] + bb[s]
            mem = beta * mem + hb - spk * _THR
            spk = (mem > _THR).astype(jnp.float32)
            spk_ref[tt * _S + s] = spk
            mem_ref[tt * _S + s] = mem
            acc = acc + spk
        return (mem, spk, acc)

    spk0 = (mem_s[...] > _THR).astype(jnp.float32)
    mem1, _, acc1 = jax.lax.fori_loop(
        0, _TC, body, (mem_s[...], spk0, acc_s[...]))
    mem_s[...] = mem1
    acc_s[...] = acc1

    @pl.when(tc == _NTC - 1)
    def _():
        cnt_ref[0] = acc1


def kernel(x, conv_w, conv_b, gamma, bn_beta, lif_beta):
    x2 = x.reshape(_N, _C)

    parts = pl.pallas_call(
        _stats_body,
        grid=(2, _NRC // 2),
        in_specs=[
            pl.BlockSpec((_RC, _C), lambda p, r: (p * (_NRC // 2) + r, 0)),
            pl.BlockSpec((_OUT, _C), lambda p, r: (0, 0)),
            pl.BlockSpec((_S, _H), lambda p, r: (0, 0)),
        ],
        out_specs=pl.BlockSpec((1, 1, _S, 2, _H),
                               lambda p, r: (p, r, 0, 0, 0)),
        out_shape=jax.ShapeDtypeStruct((2, _NRC // 2, _S, 2, _H),
                                       jnp.float32),
        compiler_params=pltpu.CompilerParams(
            dimension_semantics=("parallel", "arbitrary")),
        name="bn_stats",
    )(x2, conv_w, conv_b.reshape(_S, _H))

    parts = parts.reshape(_NRC, _S, 2, _H)
    w4 = conv_w.astype(jnp.bfloat16).reshape(_S, _H, _C)
    cb4 = conv_b.reshape(_S, _H)
    ii = jnp.arange(_B * _TC)
    perm = (((ii % _B) * _TC + ii // _B)[:, None] == ii[None, :]
            ).astype(jnp.bfloat16)
    g4 = gamma.reshape(_S, _H)
    bb4 = bn_beta.reshape(_S, _H)
    beta2 = jnp.reshape(lif_beta, (1, 1))

    spk_rec, mem_rec, cnt = pl.pallas_call(
        _scan_body,
        grid=(2, _NTC),
        in_specs=[
            pl.BlockSpec((_B, _TC, _C), lambda hh, t: (0, t, 0)),
            pl.BlockSpec((_B * _TC, _B * _TC), lambda hh, t: (0, 0)),
            pl.BlockSpec((_S, _HH, _C), lambda hh, t: (0, hh, 0)),
            pl.BlockSpec((_S, _HH), lambda hh, t: (0, hh)),
            pl.BlockSpec((_NRC, _S, 2, _HH), lambda hh, t: (0, 0, 0, hh)),
            pl.BlockSpec((_S, _HH), lambda hh, t: (0, hh)),
            pl.BlockSpec((_S, _HH), lambda hh, t: (0, hh)),
            pl.BlockSpec(memory_space=pltpu.SMEM),
        ],
        out_specs=[
            pl.BlockSpec((_TC * _S, _B, _HH), lambda hh, t: (t, 0, hh)),
            pl.BlockSpec((_TC * _S, _B, _HH), lambda hh, t: (t, 0, hh)),
            pl.BlockSpec((1, _B, _HH), lambda hh, t: (hh, 0, 0)),
        ],
        out_shape=[
            jax.ShapeDtypeStruct((_T * _S, _B, _H), jnp.float32),
            jax.ShapeDtypeStruct((_T * _S, _B, _H), jnp.float32),
            jax.ShapeDtypeStruct((2, _B, _HH), jnp.float32),
        ],
        scratch_shapes=[pltpu.VMEM((_S, _TC, _B, _HH), jnp.float32),
                        pltpu.VMEM((_B, _HH), jnp.float32),
                        pltpu.VMEM((_B, _HH), jnp.float32)],
        compiler_params=pltpu.CompilerParams(
            dimension_semantics=("parallel", "arbitrary"),
            vmem_limit_bytes=50 * 1024 * 1024),
        name="lif_gemm_scan",
    )(x, perm, w4, cb4, parts, g4, bb4, beta2)

    sum_spks = cnt.sum()
    return spk_rec, mem_rec, sum_spks
